# Initial kernel scaffold; baseline (speedup 1.0000x reference)
import jax
import jax.numpy as jnp
from jax.experimental import pallas as pl


def kernel(x, edge_index, W_m1, b_m1, W_m2, b_m2, W_l1, b_l1, W_l2, b_l2):
    raise NotImplementedError("write your pallas kernel here")


# speedup vs baseline: 46.8114x; 46.8114x over previous
import functools

import jax
import jax.numpy as jnp
from jax import lax
from jax.experimental import pallas as pl
from jax.experimental.pallas import tpu as pltpu
from jax.experimental.pallas import tpu_sc as plsc

N_NODES = 10000
N_EDGES = 320000
D_FEAT = 128
F = 4

NP = 10240
NP4 = NP * F
N_TILES = 32
EP = 327680
ET = EP // N_TILES
NV = ET // 16
SL1 = NP // 16
ROUNDS = 4
RW = NP4 // ROUNDS
SLR = RW // 16

_SC_PARAMS = pltpu.CompilerParams(
    use_tc_tiling_on_sc=False, needs_layout_passes=False
)


def _sc_mesh():
    return plsc.VectorSubcoreMesh(core_axis_name="c", subcore_axis_name="s")


@functools.lru_cache(maxsize=None)
def _make_deg_kernel():
    @functools.partial(
        pl.kernel,
        out_type=jax.ShapeDtypeStruct((2, NP), jnp.float32),
        mesh=_sc_mesh(),
        scratch_types=[
            pltpu.VMEM((ET,), jnp.int32),
            pltpu.VMEM((NP,), jnp.float32),
            pltpu.VMEM_SHARED((16, NP), jnp.float32),
        ],
        compiler_params=_SC_PARAMS,
    )
    def deg_kernel(dst_hbm, zeros_hbm, out_hbm, dst_v, acc_v, sp_sh):
        c = lax.axis_index("c")
        s = lax.axis_index("s")
        w = c * 16 + s
        pltpu.sync_copy(dst_hbm.at[pl.ds(w * ET, ET)], dst_v)
        pltpu.sync_copy(zeros_hbm, acc_v)
        ones16 = jnp.ones((16,), jnp.float32)

        def count(i, carry):
            off = pl.multiple_of(i * 16, 16)
            d16 = dst_v[pl.ds(off, 16)]
            plsc.addupdate_scatter(acc_v, [d16], ones16)
            return carry

        lax.fori_loop(0, NV, count, 0)
        pltpu.sync_copy(acc_v, sp_sh.at[s])
        plsc.subcore_barrier()
        for r in range(16):
            pltpu.sync_copy(sp_sh.at[r, pl.ds(s * SL1, SL1)],
                            acc_v.at[pl.ds(r * SL1, SL1)])

        def reduce(v, carry):
            off = pl.multiple_of(v * 16, 16)
            tot = acc_v[pl.ds(off, 16)]
            for r in range(1, 16):
                tot = tot + acc_v[pl.ds(r * SL1 + off, 16)]
            acc_v[pl.ds(off, 16)] = tot
            return carry

        lax.fori_loop(0, SL1 // 16, reduce, 0)
        pltpu.sync_copy(acc_v.at[pl.ds(0, SL1)],
                        out_hbm.at[c, pl.ds(s * SL1, SL1)])

    return deg_kernel


@functools.lru_cache(maxsize=None)
def _make_prop_kernel():
    @functools.partial(
        pl.kernel,
        out_type=jax.ShapeDtypeStruct((2, NP4), jnp.float32),
        mesh=_sc_mesh(),
        scratch_types=[
            pltpu.VMEM((ET,), jnp.int32),
            pltpu.VMEM((ET,), jnp.int32),
            pltpu.VMEM((NP4,), jnp.float32),
            pltpu.VMEM((NP4,), jnp.float32),
            pltpu.VMEM_SHARED((16, RW), jnp.float32),
        ],
        compiler_params=_SC_PARAMS,
    )
    def prop_kernel(g_hbm, src_hbm, dst_hbm, zeros_hbm, out_hbm,
                    src_v, dst_v, g_v, acc_v, sp_sh):
        c = lax.axis_index("c")
        s = lax.axis_index("s")
        w = c * 16 + s
        pltpu.sync_copy(src_hbm.at[pl.ds(w * ET, ET)], src_v)
        pltpu.sync_copy(dst_hbm.at[pl.ds(w * ET, ET)], dst_v)
        pltpu.sync_copy(g_hbm, g_v)
        pltpu.sync_copy(zeros_hbm, acc_v)

        def accumulate(i, carry):
            off = pl.multiple_of(i * 16, 16)
            sb = src_v[pl.ds(off, 16)] * 4
            db = dst_v[pl.ds(off, 16)] * 4
            for k in range(F):
                vals = plsc.load_gather(g_v, [sb + k])
                plsc.addupdate_scatter(acc_v, [db + k], vals)
            return carry

        lax.fori_loop(0, NV, accumulate, 0)
        for q in range(ROUNDS):
            pltpu.sync_copy(acc_v.at[pl.ds(q * RW, RW)], sp_sh.at[s])
            plsc.subcore_barrier()
            for r in range(16):
                pltpu.sync_copy(sp_sh.at[r, pl.ds(s * SLR, SLR)],
                                g_v.at[pl.ds(r * SLR, SLR)])

            def reduce(v, carry):
                off = pl.multiple_of(v * 16, 16)
                tot = g_v[pl.ds(off, 16)]
                for r in range(1, 16):
                    tot = tot + g_v[pl.ds(r * SLR + off, 16)]
                g_v[pl.ds(16 * SLR + off, 16)] = tot
                return carry

            lax.fori_loop(0, SLR // 16, reduce, 0)
            pltpu.sync_copy(g_v.at[pl.ds(16 * SLR, SLR)],
                            out_hbm.at[c, pl.ds(q * RW + s * SLR, SLR)])
            plsc.subcore_barrier()

    return prop_kernel


def _tc_project_norm(xp, wcat, degp):
    def body(x_ref, w_ref, p_ref, g_ref, dinv_ref):
        h = jnp.dot(x_ref[...], w_ref[...], preferred_element_type=jnp.float32)
        deg = p_ref[0] + p_ref[1] + 1.0
        dinv = lax.rsqrt(deg)
        dinv_ref[...] = dinv
        g_ref[...] = dinv * h

    return pl.pallas_call(
        body,
        out_shape=(
            jax.ShapeDtypeStruct((NP, F), jnp.float32),
            jax.ShapeDtypeStruct((NP, 1), jnp.float32),
        ),
    )(xp, wcat, degp)


def _tc_mid(p1, g0, dinv, wblk, b1):
    def body(p_ref, g_ref, d_ref, w_ref, b_ref, g1_ref):
        dinv = d_ref[...]
        out1 = dinv * (p_ref[0] + p_ref[1] + g_ref[...]) + b_ref[...]
        h1 = jnp.dot(out1, w_ref[...], preferred_element_type=jnp.float32)
        g1_ref[...] = dinv * h1

    return pl.pallas_call(
        body,
        out_shape=jax.ShapeDtypeStruct((NP, F), jnp.float32),
    )(p1, g0, dinv, wblk, b1)


def _tc_final(p2, g1, dinv, b2):
    def body(p_ref, g_ref, d_ref, b_ref, o_ref):
        o_ref[...] = d_ref[...] * (p_ref[0] + p_ref[1] + g_ref[...]) + b_ref[...]

    return pl.pallas_call(
        body,
        out_shape=jax.ShapeDtypeStruct((NP, F), jnp.float32),
    )(p2, g1, dinv, b2)


def kernel(x, edge_index, W_m1, b_m1, W_m2, b_m2, W_l1, b_l1, W_l2, b_l2):
    src = edge_index[0]
    dst = edge_index[1]
    pad = jnp.full((EP - N_EDGES,), N_NODES, dtype=jnp.int32)
    srcp = jnp.concatenate([src, pad])
    dstp = jnp.concatenate([dst, pad])
    xp = jnp.pad(x, ((0, NP - N_NODES), (0, 0)))

    wcat = jnp.concatenate([W_m1, W_l1], axis=1)
    wblk = jnp.zeros((F, F), jnp.float32)
    wblk = wblk.at[:2, :2].set(W_m2).at[2:, 2:].set(W_l2)
    b1 = jnp.concatenate([b_m1, b_l1]).reshape(1, F)
    b2 = jnp.concatenate([b_m2, b_l2]).reshape(1, F)
    zeros1 = jnp.zeros((NP,), jnp.float32)
    zeros4 = jnp.zeros((NP4,), jnp.float32)

    deg_k = _make_deg_kernel()
    prop_k = _make_prop_kernel()

    degp = deg_k(dstp, zeros1).reshape(2, NP, 1)
    g0, dinv = _tc_project_norm(xp, wcat, degp)
    p1 = prop_k(g0.reshape(-1), srcp, dstp, zeros4).reshape(2, NP, F)
    g1 = _tc_mid(p1, g0, dinv, wblk, b1)
    p2 = prop_k(g1.reshape(-1), srcp, dstp, zeros4).reshape(2, NP, F)
    out = _tc_final(p2, g1, dinv, b2)
    return out[:N_NODES]


# speedup vs baseline: 48.8302x; 1.0431x over previous
import functools

import jax
import jax.numpy as jnp
from jax import lax
from jax.experimental import pallas as pl
from jax.experimental.pallas import tpu as pltpu
from jax.experimental.pallas import tpu_sc as plsc

N_NODES = 10000
N_EDGES = 320000
D_FEAT = 128
F = 4

NP = 10240
NP4 = NP * F
N_TILES = 32
EP = 327680
ET = EP // N_TILES
NV = ET // 16
SL1 = NP // 16
ROUNDS = 2
RW = NP4 // ROUNDS
SLR = RW // 16

_SC_PARAMS = pltpu.CompilerParams(
    use_tc_tiling_on_sc=False, needs_layout_passes=False
)


def _sc_mesh():
    return plsc.VectorSubcoreMesh(core_axis_name="c", subcore_axis_name="s")


@functools.lru_cache(maxsize=None)
def _make_deg_kernel():
    @functools.partial(
        pl.kernel,
        out_type=jax.ShapeDtypeStruct((2, NP), jnp.float32),
        mesh=_sc_mesh(),
        scratch_types=[
            pltpu.VMEM((ET,), jnp.int32),
            pltpu.VMEM((NP,), jnp.float32),
            pltpu.VMEM_SHARED((16, NP), jnp.float32),
            pltpu.SemaphoreType.DMA,
        ],
        compiler_params=_SC_PARAMS,
    )
    def deg_kernel(dst_hbm, zeros_hbm, out_hbm, dst_v, acc_v, sp_sh, sem):
        c = lax.axis_index("c")
        s = lax.axis_index("s")
        w = c * 16 + s
        cps = [
            pltpu.async_copy(dst_hbm.at[pl.ds(w * ET, ET)], dst_v, sem),
            pltpu.async_copy(zeros_hbm, acc_v, sem),
        ]
        for cp in cps:
            cp.wait()
        ones16 = jnp.ones((16,), jnp.float32)

        def count(i, carry):
            for u in range(4):
                off = pl.multiple_of(i * 64 + u * 16, 16)
                d16 = dst_v[pl.ds(off, 16)]
                plsc.addupdate_scatter(acc_v, [d16], ones16)
            return carry

        lax.fori_loop(0, NV // 4, count, 0)
        pltpu.sync_copy(acc_v, sp_sh.at[s])
        plsc.subcore_barrier()
        for r in range(16):
            pltpu.sync_copy(sp_sh.at[r, pl.ds(s * SL1, SL1)],
                            acc_v.at[pl.ds(r * SL1, SL1)])

        def reduce(v, carry):
            off = pl.multiple_of(v * 16, 16)
            tot = acc_v[pl.ds(off, 16)]
            for r in range(1, 16):
                tot = tot + acc_v[pl.ds(r * SL1 + off, 16)]
            acc_v[pl.ds(off, 16)] = tot
            return carry

        lax.fori_loop(0, SL1 // 16, reduce, 0)
        pltpu.sync_copy(acc_v.at[pl.ds(0, SL1)],
                        out_hbm.at[c, pl.ds(s * SL1, SL1)])

    return deg_kernel


@functools.lru_cache(maxsize=None)
def _make_prop_kernel():
    @functools.partial(
        pl.kernel,
        out_type=jax.ShapeDtypeStruct((2, NP4), jnp.float32),
        mesh=_sc_mesh(),
        scratch_types=[
            pltpu.VMEM((ET,), jnp.int32),
            pltpu.VMEM((ET,), jnp.int32),
            pltpu.VMEM((NP4,), jnp.float32),
            pltpu.VMEM((NP4,), jnp.float32),
            pltpu.VMEM_SHARED((16, RW), jnp.float32),
            pltpu.SemaphoreType.DMA,
        ],
        compiler_params=_SC_PARAMS,
    )
    def prop_kernel(g_hbm, src_hbm, dst_hbm, zeros_hbm, out_hbm,
                    src_v, dst_v, g_v, acc_v, sp_sh, sem):
        c = lax.axis_index("c")
        s = lax.axis_index("s")
        w = c * 16 + s
        cps = [
            pltpu.async_copy(src_hbm.at[pl.ds(w * ET, ET)], src_v, sem),
            pltpu.async_copy(dst_hbm.at[pl.ds(w * ET, ET)], dst_v, sem),
            pltpu.async_copy(g_hbm, g_v, sem),
            pltpu.async_copy(zeros_hbm, acc_v, sem),
        ]
        for cp in cps:
            cp.wait()

        def accumulate(i, carry):
            for u in range(4):
                off = pl.multiple_of(i * 64 + u * 16, 16)
                sb = src_v[pl.ds(off, 16)] * 4
                db = dst_v[pl.ds(off, 16)] * 4
                for k in range(F):
                    vals = plsc.load_gather(g_v, [sb + k])
                    plsc.addupdate_scatter(acc_v, [db + k], vals)
            return carry

        lax.fori_loop(0, NV // 4, accumulate, 0)
        for q in range(ROUNDS):
            pltpu.sync_copy(acc_v.at[pl.ds(q * RW, RW)], sp_sh.at[s])
            plsc.subcore_barrier()
            for r in range(16):
                pltpu.sync_copy(sp_sh.at[r, pl.ds(s * SLR, SLR)],
                                g_v.at[pl.ds(r * SLR, SLR)])

            def reduce(v, carry):
                off = pl.multiple_of(v * 16, 16)
                tot = g_v[pl.ds(off, 16)]
                for r in range(1, 16):
                    tot = tot + g_v[pl.ds(r * SLR + off, 16)]
                g_v[pl.ds(16 * SLR + off, 16)] = tot
                return carry

            lax.fori_loop(0, SLR // 16, reduce, 0)
            pltpu.sync_copy(g_v.at[pl.ds(16 * SLR, SLR)],
                            out_hbm.at[c, pl.ds(q * RW + s * SLR, SLR)])
            plsc.subcore_barrier()

    return prop_kernel


def _tc_project_norm(xp, wcat, degp):
    def body(x_ref, w_ref, p_ref, g_ref, dinv_ref):
        h = jnp.dot(x_ref[...], w_ref[...], preferred_element_type=jnp.float32)
        deg = p_ref[0] + p_ref[1] + 1.0
        dinv = lax.rsqrt(deg)
        dinv_ref[...] = dinv
        g_ref[...] = dinv * h

    return pl.pallas_call(
        body,
        out_shape=(
            jax.ShapeDtypeStruct((NP, F), jnp.float32),
            jax.ShapeDtypeStruct((NP, 1), jnp.float32),
        ),
    )(xp, wcat, degp)


def _tc_mid(p1, g0, dinv, wblk, b1):
    def body(p_ref, g_ref, d_ref, w_ref, b_ref, g1_ref):
        dinv = d_ref[...]
        out1 = dinv * (p_ref[0] + p_ref[1] + g_ref[...]) + b_ref[...]
        h1 = jnp.dot(out1, w_ref[...], preferred_element_type=jnp.float32)
        g1_ref[...] = dinv * h1

    return pl.pallas_call(
        body,
        out_shape=jax.ShapeDtypeStruct((NP, F), jnp.float32),
    )(p1, g0, dinv, wblk, b1)


def _tc_final(p2, g1, dinv, b2):
    def body(p_ref, g_ref, d_ref, b_ref, o_ref):
        o_ref[...] = d_ref[...] * (p_ref[0] + p_ref[1] + g_ref[...]) + b_ref[...]

    return pl.pallas_call(
        body,
        out_shape=jax.ShapeDtypeStruct((NP, F), jnp.float32),
    )(p2, g1, dinv, b2)


def kernel(x, edge_index, W_m1, b_m1, W_m2, b_m2, W_l1, b_l1, W_l2, b_l2):
    src = edge_index[0]
    dst = edge_index[1]
    pad = jnp.full((EP - N_EDGES,), N_NODES, dtype=jnp.int32)
    srcp = jnp.concatenate([src, pad])
    dstp = jnp.concatenate([dst, pad])
    xp = jnp.pad(x, ((0, NP - N_NODES), (0, 0)))

    wcat = jnp.concatenate([W_m1, W_l1], axis=1)
    wblk = jnp.zeros((F, F), jnp.float32)
    wblk = wblk.at[:2, :2].set(W_m2).at[2:, 2:].set(W_l2)
    b1 = jnp.concatenate([b_m1, b_l1]).reshape(1, F)
    b2 = jnp.concatenate([b_m2, b_l2]).reshape(1, F)
    zeros1 = jnp.zeros((NP,), jnp.float32)
    zeros4 = jnp.zeros((NP4,), jnp.float32)

    deg_k = _make_deg_kernel()
    prop_k = _make_prop_kernel()

    degp = deg_k(dstp, zeros1).reshape(2, NP, 1)
    g0, dinv = _tc_project_norm(xp, wcat, degp)
    p1 = prop_k(g0.reshape(-1), srcp, dstp, zeros4).reshape(2, NP, F)
    g1 = _tc_mid(p1, g0, dinv, wblk, b1)
    p2 = prop_k(g1.reshape(-1), srcp, dstp, zeros4).reshape(2, NP, F)
    out = _tc_final(p2, g1, dinv, b2)
    return out[:N_NODES]
